# baseline (speedup 1.0000x reference)
import functools

import jax
import jax.numpy as jnp
from jax import lax
from jax.experimental import pallas as pl
from jax.experimental.pallas import tpu as pltpu
from jax.experimental.pallas import tpu_sc as plsc

_B_TILE = 2048
_ENSEMBLE = 8


def _decode_kernel(x_ref, w_ref, h_ref, d_ref, idx_ref):
    x = x_ref[...]
    h = h_ref[...]
    det = h.shape[1]
    d_all = jax.nn.sigmoid(
        jnp.dot(x, w_ref[...], preferred_element_type=jnp.float32))
    best_i = None
    best_crc = None
    for i in range(_ENSEMBLE):
        d = d_all[:, i * det:(i + 1) * det]
        hm = jax.lax.dot_general(
            d, h, (((1,), (1,)), ((), ())),
            preferred_element_type=jnp.float32)
        m2 = hm - 2.0 * jnp.floor(hm * 0.5)
        crc = jnp.sum(m2, axis=1, keepdims=True)
        if i == 0:
            best_crc = crc
            best_i = jnp.zeros(crc.shape, jnp.int32)
        else:
            take = crc < best_crc
            best_crc = jnp.where(take, crc, best_crc)
            best_i = jnp.where(take, i, best_i)
    d_ref[...] = d_all
    row = (jax.lax.broadcasted_iota(jnp.int32, best_i.shape, 0)
           + pl.program_id(0) * _B_TILE)
    idx_ref[...] = row * _ENSEMBLE + best_i


def _sc_gather(table, idx):
    rows, det = table.shape
    b = idx.shape[0]
    info = plsc.get_sparse_core_info()
    nc, ns = info.num_cores, info.num_subcores
    nw = nc * ns
    b_per_w = b // nw
    mesh = plsc.VectorSubcoreMesh(core_axis_name="c", subcore_axis_name="s")

    @functools.partial(
        pl.kernel, mesh=mesh,
        out_type=jax.ShapeDtypeStruct((b, det), jnp.float32),
        scratch_types=[
            pltpu.VMEM((b_per_w,), jnp.int32),
            pltpu.VMEM((b_per_w, det), jnp.float32),
            pltpu.SemaphoreType.DMA,
        ],
    )
    def k(table_hbm, idx_hbm, out_hbm, idx_v, rows_v, sem):
        wid = lax.axis_index("s") * nc + lax.axis_index("c")
        base = wid * b_per_w
        pltpu.sync_copy(idx_hbm.at[pl.ds(base, b_per_w)], idx_v)
        pltpu.async_copy(table_hbm.at[idx_v], rows_v, sem).wait()
        pltpu.sync_copy(rows_v, out_hbm.at[pl.ds(base, b_per_w)])

    return k(table, idx)


def kernel(x, W, code_h_outer):
    batch, in_len = x.shape
    ens, _, det = W.shape
    h_rows = code_h_outer.shape[0]
    w_flat = W.transpose(1, 0, 2).reshape(in_len, ens * det)
    d_all, gidx = pl.pallas_call(
        _decode_kernel,
        grid=(batch // _B_TILE,),
        in_specs=[
            pl.BlockSpec((_B_TILE, in_len), lambda i: (i, 0)),
            pl.BlockSpec((in_len, ens * det), lambda i: (0, 0)),
            pl.BlockSpec((h_rows, det), lambda i: (0, 0)),
        ],
        out_specs=[
            pl.BlockSpec((_B_TILE, ens * det), lambda i: (i, 0)),
            pl.BlockSpec((_B_TILE, 1), lambda i: (i, 0)),
        ],
        out_shape=[
            jax.ShapeDtypeStruct((batch, ens * det), jnp.float32),
            jax.ShapeDtypeStruct((batch, 1), jnp.int32),
        ],
        compiler_params=pltpu.CompilerParams(
            dimension_semantics=("parallel",)),
    )(x, w_flat, code_h_outer)
    table = d_all.reshape(batch * ens, det)
    return _sc_gather(table, gidx.reshape(batch))


# speedup vs baseline: 3.3382x; 3.3382x over previous
import jax
import jax.numpy as jnp
from jax.experimental import pallas as pl
from jax.experimental.pallas import tpu as pltpu

_B_TILE = 4096
_ENSEMBLE = 8


def _fused_kernel(x_ref, w_ref, h_ref, out_ref):
    x = x_ref[...]
    h = h_ref[...]
    det = h.shape[1]
    d_all = jax.nn.sigmoid(
        jnp.dot(x, w_ref[...], preferred_element_type=jnp.float32))
    best = None
    best_crc = None
    for i in range(_ENSEMBLE):
        d = d_all[:, i * det:(i + 1) * det]
        hm = jax.lax.dot_general(
            d, h, (((1,), (1,)), ((), ())),
            preferred_element_type=jnp.float32)
        m2 = hm - jnp.floor(hm)
        crc = jnp.sum(m2, axis=1, keepdims=True)
        if i == 0:
            best, best_crc = d, crc
        else:
            take = crc < best_crc
            best = jnp.where(take, d, best)
            best_crc = jnp.where(take, crc, best_crc)
    out_ref[...] = best


def kernel(x, W, code_h_outer):
    batch, in_len = x.shape
    ens, _, det = W.shape
    h_rows = code_h_outer.shape[0]
    w_flat = W.transpose(1, 0, 2).reshape(in_len, ens * det)
    return pl.pallas_call(
        _fused_kernel,
        grid=(batch // _B_TILE,),
        in_specs=[
            pl.BlockSpec((_B_TILE, in_len), lambda i: (i, 0)),
            pl.BlockSpec((in_len, ens * det), lambda i: (0, 0)),
            pl.BlockSpec((h_rows, det), lambda i: (0, 0)),
        ],
        out_specs=pl.BlockSpec((_B_TILE, det), lambda i: (i, 0)),
        out_shape=jax.ShapeDtypeStruct((batch, det), jnp.float32),
        compiler_params=pltpu.CompilerParams(
            dimension_semantics=("parallel",)),
    )(x, w_flat, code_h_outer * 0.5)


# speedup vs baseline: 3.3501x; 1.0036x over previous
import jax
import jax.numpy as jnp
from jax.experimental import pallas as pl
from jax.experimental.pallas import tpu as pltpu

_B_TILE = 2048
_ENSEMBLE = 8


def _fused_kernel(x_ref, w_ref, h_ref, out_ref):
    x = x_ref[...]
    h = h_ref[...]
    det = h.shape[1]
    d_all = jax.nn.sigmoid(
        jnp.dot(x, w_ref[...], preferred_element_type=jnp.float32))
    best = None
    best_crc = None
    for i in range(_ENSEMBLE):
        d = d_all[:, i * det:(i + 1) * det]
        hm = jax.lax.dot_general(
            d, h, (((1,), (1,)), ((), ())),
            preferred_element_type=jnp.float32)
        m2 = hm - jnp.floor(hm)
        crc = jnp.sum(m2, axis=1, keepdims=True)
        if i == 0:
            best, best_crc = d, crc
        else:
            take = crc < best_crc
            best = jnp.where(take, d, best)
            best_crc = jnp.where(take, crc, best_crc)
    out_ref[...] = best


def kernel(x, W, code_h_outer):
    batch, in_len = x.shape
    ens, _, det = W.shape
    h_rows = code_h_outer.shape[0]
    w_flat = W.transpose(1, 0, 2).reshape(in_len, ens * det)
    return pl.pallas_call(
        _fused_kernel,
        grid=(batch // _B_TILE,),
        in_specs=[
            pl.BlockSpec((_B_TILE, in_len), lambda i: (i, 0)),
            pl.BlockSpec((in_len, ens * det), lambda i: (0, 0)),
            pl.BlockSpec((h_rows, det), lambda i: (0, 0)),
        ],
        out_specs=pl.BlockSpec((_B_TILE, det), lambda i: (i, 0)),
        out_shape=jax.ShapeDtypeStruct((batch, det), jnp.float32),
        compiler_params=pltpu.CompilerParams(
            dimension_semantics=("parallel",)),
    )(x, w_flat, code_h_outer * 0.5)
